# initial kernel scaffold (unmeasured)
import jax
import jax.numpy as jnp
from jax import lax
from jax.experimental import pallas as pl
from jax.experimental.pallas import tpu as pltpu


def kernel(
    x,
):
    def body(*refs):
        pass

    out_shape = jax.ShapeDtypeStruct(..., jnp.float32)
    return pl.pallas_call(body, out_shape=out_shape)(...)


# baseline (device time: 98574 ns/iter reference)
import jax
import jax.numpy as jnp
from jax import lax
from jax.experimental import pallas as pl
from jax.experimental.pallas import tpu as pltpu

K = 8


def kernel(x):
    _, m, n2 = x.shape
    n_half = n2 // 2
    rows = m // K

    def body(x_ref, out_ref, load_buf, send_full, recv_full,
             load_sems, send_sems, recv_sems):
        my_x = lax.axis_index("x")
        my_y = lax.axis_index("y")
        my_z = lax.axis_index("z")
        other = 1 - my_x

        barrier = pltpu.get_barrier_semaphore()
        pl.semaphore_signal(
            barrier,
            inc=1,
            device_id=(other, my_y, my_z),
            device_id_type=pl.DeviceIdType.MESH,
        )
        pl.semaphore_wait(barrier, 1)

        def rdma_for_chunk(c):
            r = pl.ds(c * rows, rows)
            return pltpu.make_async_remote_copy(
                src_ref=send_full.at[r, :],
                dst_ref=recv_full.at[r, :],
                send_sem=send_sems.at[c],
                recv_sem=recv_sems.at[c],
                device_id=(other, my_y, my_z),
                device_id_type=pl.DeviceIdType.MESH,
            )

        def send_phase(their_col):
            for c in range(K):
                slot = c % 2
                r = pl.ds(c * rows, rows)
                cp = pltpu.make_async_copy(
                    x_ref.at[0, r, their_col],
                    load_buf.at[slot],
                    load_sems.at[slot],
                )
                cp.start()
                cp.wait()
                send_full[r, :] = load_buf[slot].astype(jnp.bfloat16)
                rdma_for_chunk(c).start()

        def local_phase(my_col):
            for c in range(K):
                slot = c % 2
                r = pl.ds(c * rows, rows)
                cp = pltpu.make_async_copy(
                    x_ref.at[0, r, my_col],
                    load_buf.at[slot],
                    load_sems.at[slot],
                )
                cp.start()
                cp.wait()
                out_ref[r, :] = load_buf[slot].astype(jnp.bfloat16)

        lo = slice(0, n_half)
        hi = slice(n_half, n2)

        @pl.when(my_x == 0)
        def _():
            send_phase(hi)
            local_phase(lo)

        @pl.when(my_x == 1)
        def _():
            send_phase(lo)
            local_phase(hi)

        for c in range(K):
            r = pl.ds(c * rows, rows)
            rdma = rdma_for_chunk(c)
            rdma.wait_send()
            rdma.wait_recv()
            out_ref[r, :] = out_ref[r, :] + recv_full[r, :]

    return pl.pallas_call(
        body,
        out_shape=jax.ShapeDtypeStruct((m, n_half), jnp.bfloat16),
        in_specs=[pl.BlockSpec(memory_space=pl.ANY)],
        out_specs=pl.BlockSpec(memory_space=pltpu.VMEM),
        scratch_shapes=[
            pltpu.VMEM((2, rows, n_half), jnp.float32),
            pltpu.VMEM((m, n_half), jnp.bfloat16),
            pltpu.VMEM((m, n_half), jnp.bfloat16),
            pltpu.SemaphoreType.DMA((2,)),
            pltpu.SemaphoreType.DMA((K,)),
            pltpu.SemaphoreType.DMA((K,)),
        ],
        compiler_params=pltpu.CompilerParams(collective_id=0),
    )(x)


# device time: 71053 ns/iter; 1.3873x vs baseline; 1.3873x over previous
import jax
import jax.numpy as jnp
from jax import lax
from jax.experimental import pallas as pl
from jax.experimental.pallas import tpu as pltpu

NS = 16


def kernel(x):
    _, m, n2 = x.shape
    n_half = n2 // 2
    mh = m // 2
    rows = mh // NS

    def body(x_ref, out_ref, ld_buf, x_send, x_recv,
             ld_sems, xs_send, xs_recv, ys_send, ys_recv):
        my_x = lax.axis_index("x")
        my_y = lax.axis_index("y")
        my_z = lax.axis_index("z")
        x_nbr = (1 - my_x, my_y, my_z)
        y_nbr = (my_x, 1 - my_y, my_z)

        p1_off = my_y * mh

        barrier = pltpu.get_barrier_semaphore()
        for nbr in (x_nbr, y_nbr):
            pl.semaphore_signal(
                barrier, inc=1, device_id=nbr,
                device_id_type=pl.DeviceIdType.MESH,
            )
        pl.semaphore_wait(barrier, 2)

        def rdma_x(s):
            r = pl.ds(s * rows, rows)
            return pltpu.make_async_remote_copy(
                src_ref=x_send.at[r, :],
                dst_ref=x_recv.at[r, :],
                send_sem=xs_send.at[s],
                recv_sem=xs_recv.at[s],
                device_id=x_nbr,
                device_id_type=pl.DeviceIdType.MESH,
            )

        def rdma_y(s):
            r = pl.ds(p1_off + s * rows, rows)
            return pltpu.make_async_remote_copy(
                src_ref=out_ref.at[r, :],
                dst_ref=out_ref.at[r, :],
                send_sem=ys_send.at[s],
                recv_sem=ys_recv.at[s],
                device_id=y_nbr,
                device_id_type=pl.DeviceIdType.MESH,
            )

        def go(my_col, their_col):
            for s in range(NS):
                slot = s % 2
                cp = pltpu.make_async_copy(
                    x_ref.at[0, pl.ds(p1_off + s * rows, rows), their_col],
                    ld_buf.at[slot],
                    ld_sems.at[slot],
                )
                cp.start()
                cp.wait()
                x_send[pl.ds(s * rows, rows), :] = (
                    ld_buf[slot].astype(jnp.bfloat16))
                rdma_x(s).start()

            for s in range(NS):
                slot = s % 2
                cp = pltpu.make_async_copy(
                    x_ref.at[0, pl.ds(p1_off + s * rows, rows), my_col],
                    ld_buf.at[slot],
                    ld_sems.at[slot],
                )
                cp.start()
                cp.wait()
                rdma_x(s).wait_recv()
                out_ref[pl.ds(p1_off + s * rows, rows), :] = (
                    ld_buf[slot].astype(jnp.bfloat16)
                    + x_recv[pl.ds(s * rows, rows), :])
                rdma_y(s).start()

        lo = slice(0, n_half)
        hi = slice(n_half, n2)

        @pl.when(my_x == 0)
        def _():
            go(lo, hi)

        @pl.when(my_x == 1)
        def _():
            go(hi, lo)

        for s in range(NS):
            r = rdma_y(s)
            r.wait_recv()
            r.wait_send()
            rdma_x(s).wait_send()

    return pl.pallas_call(
        body,
        out_shape=jax.ShapeDtypeStruct((m, n_half), jnp.bfloat16),
        in_specs=[pl.BlockSpec(memory_space=pl.ANY)],
        out_specs=pl.BlockSpec(memory_space=pltpu.VMEM),
        scratch_shapes=[
            pltpu.VMEM((2, rows, n_half), jnp.float32),
            pltpu.VMEM((mh, n_half), jnp.bfloat16),
            pltpu.VMEM((mh, n_half), jnp.bfloat16),
            pltpu.SemaphoreType.DMA((2,)),
            pltpu.SemaphoreType.DMA((NS,)),
            pltpu.SemaphoreType.DMA((NS,)),
            pltpu.SemaphoreType.DMA((NS,)),
            pltpu.SemaphoreType.DMA((NS,)),
        ],
        compiler_params=pltpu.CompilerParams(collective_id=0),
    )(x)


# device time: 61104 ns/iter; 1.6132x vs baseline; 1.1628x over previous
import jax
import jax.numpy as jnp
from jax import lax
from jax.experimental import pallas as pl
from jax.experimental.pallas import tpu as pltpu

NS = 16
LAG = 4
NC = 2


def kernel(x):
    _, m, n2 = x.shape
    n_half = n2 // 2
    mh = m // 2
    rows = mh // NS
    crows = mh // NC
    spc = NS // NC

    def body(x_ref, out_ref, ld_buf, big_buf, x_send, x_recv, y_recv,
             ld_sems, big_sems, xs_send, xs_recv, ys_send, ys_recv):
        my_x = lax.axis_index("x")
        my_y = lax.axis_index("y")
        my_z = lax.axis_index("z")
        x_nbr = (1 - my_x, my_y, my_z)
        y_nbr = (my_x, 1 - my_y, my_z)

        p1_off = my_y * mh
        p2_off = (1 - my_y) * mh

        barrier = pltpu.get_barrier_semaphore()
        for nbr in (x_nbr, y_nbr):
            pl.semaphore_signal(
                barrier, inc=1, device_id=nbr,
                device_id_type=pl.DeviceIdType.MESH,
            )
        pl.semaphore_wait(barrier, 2)

        def rdma_x(s):
            r = pl.ds(s * rows, rows)
            return pltpu.make_async_remote_copy(
                src_ref=x_send.at[r, :],
                dst_ref=x_recv.at[r, :],
                send_sem=xs_send.at[s],
                recv_sem=xs_recv.at[s],
                device_id=x_nbr,
                device_id_type=pl.DeviceIdType.MESH,
            )

        def rdma_y(s):
            r = pl.ds(s * rows, rows)
            return pltpu.make_async_remote_copy(
                src_ref=x_recv.at[r, :],
                dst_ref=y_recv.at[r, :],
                send_sem=ys_send.at[s],
                recv_sem=ys_recv.at[s],
                device_id=y_nbr,
                device_id_type=pl.DeviceIdType.MESH,
            )

        def go(my_col, their_col):
            def start_load(s, slot):
                return pltpu.make_async_copy(
                    x_ref.at[0, pl.ds(p1_off + s * rows, rows), their_col],
                    ld_buf.at[slot],
                    ld_sems.at[slot],
                )

            start_load(0, 0).start()
            for s in range(NS):
                if s + 1 < NS:
                    start_load(s + 1, (s + 1) % 2).start()
                start_load(s, s % 2).wait()
                x_send[pl.ds(s * rows, rows), :] = (
                    ld_buf[s % 2].astype(jnp.bfloat16))
                rdma_x(s).start()
                b = s - LAG
                if b >= 0:
                    rdma_x(b).wait_recv()
                    rdma_y(b).start()
            for b in range(NS - LAG, NS):
                rdma_x(b).wait_recv()
                rdma_y(b).start()

            def big_load(c, slot):
                row0 = jnp.where(c < NC, p1_off + c * crows,
                                 p2_off + (c - NC) * crows)
                return pltpu.make_async_copy(
                    x_ref.at[0, pl.ds(row0, crows), my_col],
                    big_buf.at[slot],
                    big_sems.at[slot],
                )

            big_load(0, 0).start()
            for c in range(2 * NC):
                if c + 1 < 2 * NC:
                    big_load(c + 1, (c + 1) % 2).start()
                big_load(c, c % 2).wait()
                rc = pl.ds((c % NC) * crows, crows)
                if c < NC:
                    out_ref[pl.ds(p1_off + c * crows, crows), :] = (
                        big_buf[c % 2].astype(jnp.bfloat16) + x_recv[rc, :])
                else:
                    for s in range((c - NC) * spc, (c - NC + 1) * spc):
                        rdma_y(s).wait_recv()
                    out_ref[pl.ds(p2_off + (c - NC) * crows, crows), :] = (
                        big_buf[c % 2].astype(jnp.bfloat16) + y_recv[rc, :])

        lo = slice(0, n_half)
        hi = slice(n_half, n2)

        @pl.when(my_x == 0)
        def _():
            go(lo, hi)

        @pl.when(my_x == 1)
        def _():
            go(hi, lo)

        for s in range(NS):
            rdma_x(s).wait_send()
            rdma_y(s).wait_send()

    return pl.pallas_call(
        body,
        out_shape=jax.ShapeDtypeStruct((m, n_half), jnp.bfloat16),
        in_specs=[pl.BlockSpec(memory_space=pl.ANY)],
        out_specs=pl.BlockSpec(memory_space=pltpu.VMEM),
        scratch_shapes=[
            pltpu.VMEM((2, rows, n_half), jnp.float32),
            pltpu.VMEM((2, mh // NC, n_half), jnp.float32),
            pltpu.VMEM((mh, n_half), jnp.bfloat16),
            pltpu.VMEM((mh, n_half), jnp.bfloat16),
            pltpu.VMEM((mh, n_half), jnp.bfloat16),
            pltpu.SemaphoreType.DMA((2,)),
            pltpu.SemaphoreType.DMA((2,)),
            pltpu.SemaphoreType.DMA((NS,)),
            pltpu.SemaphoreType.DMA((NS,)),
            pltpu.SemaphoreType.DMA((NS,)),
            pltpu.SemaphoreType.DMA((NS,)),
        ],
        compiler_params=pltpu.CompilerParams(collective_id=0),
    )(x)
